# baseline (speedup 1.0000x reference)
import functools

import jax
import jax.numpy as jnp
from jax import lax
from jax.experimental import pallas as pl
from jax.experimental.pallas import tpu as pltpu
from jax.experimental.pallas import tpu_sc as plsc

EPS = 1e-5

_NS = 16
_CH = 512
_IDXU = _CH // 16


def _sc_mesh():
    return plsc.VectorSubcoreMesh(core_axis_name="c", subcore_axis_name="s")


def _zero_rows(buf, nrows, ncol16):
    z = jnp.zeros((16,), jnp.float32)

    @pl.loop(0, nrows)
    def _(j):
        for c in range(ncol16):
            buf[j, pl.ds(c * 16, 16)] = z


def _fill_ones(buf, nrows, ncol16):
    o = jnp.ones((16,), jnp.float32)

    @pl.loop(0, nrows)
    def _(j):
        for c in range(ncol16):
            buf[j, pl.ds(c * 16, 16)] = o


def _sc_gcn_agg_maker(half, colw, colgroups, e_pad, with_gather,
                      wb_chunk, wb_nch):
    half_pad = _NS * wb_chunk * wb_nch
    per_tile = e_pad // _NS
    n_ch = per_tile // _CH
    zch = -(-max(half + 8, half_pad) // (_NS * _CH))
    acc_rows = zch * _NS * _CH
    dump = half

    scratch = [
        pltpu.VMEM((_CH,), jnp.int32),
        pltpu.VMEM((_CH,), jnp.int32),
        pltpu.VMEM((_CH,), jnp.int32),
        pltpu.VMEM((_CH,), jnp.int32),
        pltpu.VMEM((_CH, colw), jnp.float32),
        pltpu.VMEM((wb_chunk, colw), jnp.float32),
        pltpu.VMEM_SHARED((acc_rows, colw), jnp.float32),
        pltpu.SemaphoreType.DMA,
    ]

    def body(*refs):
        if with_gather:
            (tbl, srch, dsth, out, srcv, dstv, gidx, ldx, rows, stag, acc,
             sem) = refs
        else:
            (dsth, out, srcv, dstv, gidx, ldx, rows, stag, acc, sem) = refs
            tbl = None
        cid = lax.axis_index("c")
        sid = lax.axis_index("s")
        base = cid * half
        ebase = sid * per_tile

        @pl.loop(0, colgroups)
        def _cg(cg):
            _zero_rows(rows, _CH, colw // 16)

            @pl.loop(0, zch)
            def _(k):
                pltpu.sync_copy(rows, acc.at[pl.ds(sid * zch * _CH
                                                   + k * _CH, _CH)])
            plsc.subcore_barrier()
            if not with_gather:
                _fill_ones(rows, _CH, colw // 16)

            @pl.loop(0, n_ch)
            def _(ch):
                off = ebase + ch * _CH
                if with_gather:
                    pltpu.sync_copy(srch.at[pl.ds(off, _CH)], srcv)
                pltpu.sync_copy(dsth.at[pl.ds(off, _CH)], dstv)
                for k in range(_IDXU):
                    d16 = dstv[pl.ds(k * 16, 16)]
                    ok = (d16 >= base) & (d16 < base + half)
                    l16 = jnp.where(ok, d16 - base, dump)
                    ldx[pl.ds(k * 16, 16)] = l16
                    if with_gather:
                        s16 = srcv[pl.ds(k * 16, 16)]
                        g16 = s16 * colgroups + cg
                        gidx[pl.ds(k * 16, 16)] = g16
                if with_gather:
                    pltpu.async_copy(tbl.at[gidx], rows, sem).wait()
                pltpu.sync_copy(rows, acc.at[ldx], add=True)
            plsc.subcore_barrier()

            @pl.loop(0, wb_nch)
            def _(w):
                r0 = sid * wb_nch * wb_chunk + w * wb_chunk
                pltpu.sync_copy(acc.at[pl.ds(r0, wb_chunk)], stag)
                pltpu.sync_copy(stag, out.at[cg, pl.ds(cid * half_pad + r0,
                                                       wb_chunk)])
            plsc.subcore_barrier()

    kern = pl.kernel(
        body,
        out_type=jax.ShapeDtypeStruct((colgroups, 2 * half_pad, colw),
                                      jnp.float32),
        mesh=_sc_mesh(),
        scratch_types=scratch,
        compiler_params=pltpu.CompilerParams(use_tc_tiling_on_sc=False),
    )
    return kern


def sc_gcn_agg1(h1n, src_p, dst_p):
    k = _sc_gcn_agg_maker(5000, 64, 2, src_p.shape[0], True, 160, 2)
    o = k(h1n.reshape(-1, 64), src_p, dst_p)
    o = jnp.concatenate([o[:, :5000], o[:, 5120:10120]], axis=1)
    return jnp.concatenate([o[0], o[1]], axis=1)


def sc_gcn_agg2(h2n, src_p, dst_p):
    k = _sc_gcn_agg_maker(100000, 16, 8, src_p.shape[0], True, 640, 10)
    o = k(h2n.reshape(-1, 16), src_p, dst_p)
    return jnp.concatenate([o[:, :100000], o[:, 102400:202400]], axis=1)


def sc_degree(dst_p, half, wb_chunk, wb_nch):
    k = _sc_gcn_agg_maker(half, 16, 1, dst_p.shape[0], False,
                          wb_chunk, wb_nch)
    o = k(dst_p)[0]
    hp = _NS * wb_chunk * wb_nch
    return jnp.concatenate([o[:half], o[hp:hp + half]], axis=0)


def sc_segsum(h2new, l2b2):
    m = h2new.shape[0]
    chr_ = 400
    per_tile = 8000
    n_ch = per_tile // chr_

    def body(vals_hbm, idx_hbm, sums_out, cnt_out, vbuf, obuf, ibuf,
             acc_s, acc_c, sem):
        cid = lax.axis_index("c")
        sid = lax.axis_index("s")
        tg = cid * _NS + sid
        _zero_rows(vbuf, 264, 8)
        _zero_rows(obuf, 264, 1)

        @pl.when(sid == 0)
        def _():
            pltpu.sync_copy(vbuf.at[pl.ds(0, 264)], acc_s)
            pltpu.sync_copy(obuf.at[pl.ds(0, 264)], acc_c)
        plsc.subcore_barrier()
        _fill_ones(obuf, chr_, 1)

        @pl.when(tg < 25)
        def _():
            @pl.loop(0, n_ch)
            def _(ch):
                off = tg * per_tile + ch * chr_
                pltpu.sync_copy(vals_hbm.at[pl.ds(off, chr_)], vbuf)
                pltpu.sync_copy(idx_hbm.at[pl.ds(off, chr_)], ibuf)
                pltpu.sync_copy(vbuf, acc_s.at[ibuf], add=True)
                pltpu.sync_copy(obuf, acc_c.at[ibuf], add=True)
        plsc.subcore_barrier()

        @pl.when(sid == 0)
        def _():
            pltpu.sync_copy(acc_s, sums_out.at[cid])
            pltpu.sync_copy(acc_c, cnt_out.at[cid])

    kern = pl.kernel(
        body,
        out_type=[jax.ShapeDtypeStruct((2, 264, 128), jnp.float32),
                  jax.ShapeDtypeStruct((2, 264, 16), jnp.float32)],
        mesh=_sc_mesh(),
        scratch_types=[
            pltpu.VMEM((chr_, 128), jnp.float32),
            pltpu.VMEM((chr_, 16), jnp.float32),
            pltpu.VMEM((chr_,), jnp.int32),
            pltpu.VMEM_SHARED((264, 128), jnp.float32),
            pltpu.VMEM_SHARED((264, 16), jnp.float32),
            pltpu.SemaphoreType.DMA,
        ],
        compiler_params=pltpu.CompilerParams(use_tc_tiling_on_sc=False),
    )
    return kern(h2new, l2b2)


def _gelu(x):
    return 0.5 * x * (1.0 + jax.lax.erf(x * 0.7071067811865476))


def _rows_spec(blk, w):
    return pl.BlockSpec((blk, w), lambda i: (i, 0))


def _stats_spec():
    return pl.BlockSpec((8, 128), lambda i: (0, 0))


def _stats_merge(st_ref, r, x, blk):
    i = pl.program_id(0)
    mb = jnp.sum(x, axis=0, keepdims=True) * (1.0 / blk)
    m2b = jnp.sum((x - mb) ** 2, axis=0, keepdims=True)

    @pl.when(i == 0)
    def _():
        st_ref[r:r + 1] = mb
        st_ref[r + 1:r + 2] = m2b

    @pl.when(i != 0)
    def _():
        nf = i.astype(jnp.float32) * float(blk)
        mean = st_ref[r:r + 1]
        delta = mb - mean
        tot = nf + float(blk)
        st_ref[r:r + 1] = mean + delta * (float(blk) / tot)
        st_ref[r + 1:r + 2] = (st_ref[r + 1:r + 2] + m2b
                               + delta * delta * (nf * float(blk) / tot))


def _scale_rows_matmul(x_ref, w_ref, s_ref, o_ref):
    h = jnp.dot(x_ref[...], w_ref[...], preferred_element_type=jnp.float32)
    o_ref[...] = s_ref[...] * h


def tc_scale_rows_matmul(x, w, s, blk):
    n, d = x.shape
    return pl.pallas_call(
        _scale_rows_matmul,
        grid=(n // blk,),
        in_specs=[_rows_spec(blk, d), pl.BlockSpec((d, w.shape[1]), lambda i: (0, 0)),
                  _rows_spec(blk, 1)],
        out_specs=_rows_spec(blk, w.shape[1]),
        out_shape=jax.ShapeDtypeStruct((n, w.shape[1]), jnp.float32),
    )(x, w, s)


def _gcn_post(agg_ref, hn_ref, s_ref, b_ref, o_ref):
    o_ref[...] = s_ref[...] * (agg_ref[...] + hn_ref[...]) + b_ref[...]


def tc_gcn_post(agg, hn, s, b, blk):
    n, d = agg.shape
    return pl.pallas_call(
        _gcn_post,
        grid=(n // blk,),
        in_specs=[_rows_spec(blk, d), _rows_spec(blk, d), _rows_spec(blk, 1),
                  pl.BlockSpec((1, d), lambda i: (0, 0))],
        out_specs=_rows_spec(blk, d),
        out_shape=jax.ShapeDtypeStruct((n, d), jnp.float32),
    )(agg, hn, s, b.reshape(1, d))


def _p1_body(blk, hu_ref, hv_ref, w_ref, b_ref, u_ref, es_ref, st_ref):
    hu = hu_ref[...]
    hv = hv_ref[...]
    w = w_ref[...]
    b = b_ref[...]
    c1 = jnp.concatenate([hu, hv], axis=1)
    c2 = jnp.concatenate([hv, hu], axis=1)
    u1 = jnp.dot(c1, w, preferred_element_type=jnp.float32) + b
    u2 = jnp.dot(c2, w, preferred_element_type=jnp.float32) + b
    u = jnp.concatenate([u1, u2], axis=1)
    u_ref[...] = u
    es_ref[...] = hu + hv
    _stats_merge(st_ref, 0, u, blk)


def tc_p1(hu, hv, w_ne0, b_ne0, blk):
    e = hu.shape[0]
    return pl.pallas_call(
        functools.partial(_p1_body, float(blk)),
        grid=(e // blk,),
        in_specs=[_rows_spec(blk, 128), _rows_spec(blk, 128),
                  pl.BlockSpec((256, 64), lambda i: (0, 0)),
                  pl.BlockSpec((1, 64), lambda i: (0, 0))],
        out_specs=[_rows_spec(blk, 128), _rows_spec(blk, 128), _stats_spec()],
        out_shape=[jax.ShapeDtypeStruct((e, 128), jnp.float32),
                   jax.ShapeDtypeStruct((e, 128), jnp.float32),
                   jax.ShapeDtypeStruct((8, 128), jnp.float32)],
    )(hu, hv, w_ne0, b_ne0.reshape(1, 64))


def _p2_body(u_ref, s_ref, t_ref, w_ref, b_ref, o_ref):
    g = _gelu(u_ref[...] * s_ref[...] + t_ref[...])
    w = w_ref[...]
    b = b_ref[...]
    o1 = jnp.dot(g[:, :64], w, preferred_element_type=jnp.float32) + b
    o2 = jnp.dot(g[:, 64:], w, preferred_element_type=jnp.float32) + b
    o_ref[...] = 0.5 * (o1 + o2)


def tc_p2(u, s, t, w_ne1, b_ne1, blk):
    e = u.shape[0]
    return pl.pallas_call(
        _p2_body,
        grid=(e // blk,),
        in_specs=[_rows_spec(blk, 128),
                  pl.BlockSpec((1, 128), lambda i: (0, 0)),
                  pl.BlockSpec((1, 128), lambda i: (0, 0)),
                  pl.BlockSpec((64, 64), lambda i: (0, 0)),
                  pl.BlockSpec((1, 64), lambda i: (0, 0))],
        out_specs=_rows_spec(blk, 64),
        out_shape=jax.ShapeDtypeStruct((e, 64), jnp.float32),
    )(u, s.reshape(1, 128), t.reshape(1, 128), w_ne1, b_ne1.reshape(1, 64))


def _p3_body(blk, he_ref, hf_ref, ee_ref, ef_ref, w_ref, b_ref,
             v_ref, hm_ref, st_ref):
    he = he_ref[...]
    hf = hf_ref[...]
    w = w_ref[...]
    b = b_ref[...]
    v1 = jnp.dot(jnp.concatenate([hf, he], axis=1), w,
                 preferred_element_type=jnp.float32) + b
    v2 = jnp.dot(jnp.concatenate([he, hf], axis=1), w,
                 preferred_element_type=jnp.float32) + b
    v_ref[:, :128] = v1
    v_ref[:, 128:] = v2
    hm_ref[...] = 0.25 * (ee_ref[...] + ef_ref[...])
    _stats_merge(st_ref, 0, v1, blk)
    _stats_merge(st_ref, 2, v2, blk)


def tc_p3(he, hf, ee, ef, w_ef0, b_ef0, blk):
    m = he.shape[0]
    return pl.pallas_call(
        functools.partial(_p3_body, float(blk)),
        grid=(m // blk,),
        in_specs=[_rows_spec(blk, 64), _rows_spec(blk, 64),
                  _rows_spec(blk, 128), _rows_spec(blk, 128),
                  pl.BlockSpec((128, 128), lambda i: (0, 0)),
                  pl.BlockSpec((1, 128), lambda i: (0, 0))],
        out_specs=[_rows_spec(blk, 256), _rows_spec(blk, 128), _stats_spec()],
        out_shape=[jax.ShapeDtypeStruct((m, 256), jnp.float32),
                   jax.ShapeDtypeStruct((m, 128), jnp.float32),
                   jax.ShapeDtypeStruct((8, 128), jnp.float32)],
    )(he, hf, ee, ef, w_ef0, b_ef0.reshape(1, 128))


def _p4_body(blk, v_ref, s_ref, t_ref, w_ref, b_ref, h1_ref, st_ref):
    s = s_ref[...]
    t = t_ref[...]
    g1 = _gelu(v_ref[:, :128] * s[:, :128] + t[:, :128])
    g2 = _gelu(v_ref[:, 128:] * s[:, 128:] + t[:, 128:])
    w = w_ref[...]
    b = b_ref[...]
    h1a = jnp.dot(g1, w, preferred_element_type=jnp.float32) + b
    h1b = jnp.dot(g2, w, preferred_element_type=jnp.float32) + b
    h1 = 0.5 * (h1a + h1b)
    h1_ref[...] = h1
    _stats_merge(st_ref, 0, h1, blk)


def tc_p4(v, s, t, w_ef1, b_ef1, blk):
    m = v.shape[0]
    return pl.pallas_call(
        functools.partial(_p4_body, float(blk)),
        grid=(m // blk,),
        in_specs=[_rows_spec(blk, 256),
                  pl.BlockSpec((1, 256), lambda i: (0, 0)),
                  pl.BlockSpec((1, 256), lambda i: (0, 0)),
                  pl.BlockSpec((128, 128), lambda i: (0, 0)),
                  pl.BlockSpec((1, 128), lambda i: (0, 0))],
        out_specs=[_rows_spec(blk, 128), _stats_spec()],
        out_shape=[jax.ShapeDtypeStruct((m, 128), jnp.float32),
                   jax.ShapeDtypeStruct((8, 128), jnp.float32)],
    )(v, s.reshape(1, 256), t.reshape(1, 256), w_ef1, b_ef1.reshape(1, 128))


def _p5_body(h1_ref, hm_ref, s_ref, t_ref, dinv_ref, w_ref, hn_ref, h2n_ref):
    h1n = hm_ref[...] + jax.nn.relu(h1_ref[...] * s_ref[...] + t_ref[...])
    hn_ref[...] = h1n
    h2 = jnp.dot(h1n, w_ref[...], preferred_element_type=jnp.float32)
    h2n_ref[...] = dinv_ref[...] * h2


def tc_p5(h1, hm, s, t, dinv2, w_gcn2, blk):
    m = h1.shape[0]
    return pl.pallas_call(
        _p5_body,
        grid=(m // blk,),
        in_specs=[_rows_spec(blk, 128), _rows_spec(blk, 128),
                  pl.BlockSpec((1, 128), lambda i: (0, 0)),
                  pl.BlockSpec((1, 128), lambda i: (0, 0)),
                  _rows_spec(blk, 1),
                  pl.BlockSpec((128, 128), lambda i: (0, 0))],
        out_specs=[_rows_spec(blk, 128), _rows_spec(blk, 128)],
        out_shape=[jax.ShapeDtypeStruct((m, 128), jnp.float32),
                   jax.ShapeDtypeStruct((m, 128), jnp.float32)],
    )(h1, hm, s.reshape(1, 128), t.reshape(1, 128), dinv2, w_gcn2)


def _p6_body(blk, agg_ref, h2n_ref, dinv_ref, b_ref, h2_ref, st_ref):
    a = agg_ref[...]
    agg = jnp.concatenate([a[cg] for cg in range(8)], axis=1)
    h2 = dinv_ref[...] * (agg + h2n_ref[...]) + b_ref[...]
    h2_ref[...] = h2
    _stats_merge(st_ref, 0, h2, blk)


def tc_p6(agg, h2n, dinv2, b_gcn2, blk):
    m = h2n.shape[0]
    return pl.pallas_call(
        functools.partial(_p6_body, float(blk)),
        grid=(m // blk,),
        in_specs=[pl.BlockSpec((8, blk, 16), lambda i: (0, i, 0)),
                  _rows_spec(blk, 128), _rows_spec(blk, 1),
                  pl.BlockSpec((1, 128), lambda i: (0, 0))],
        out_specs=[_rows_spec(blk, 128), _stats_spec()],
        out_shape=[jax.ShapeDtypeStruct((m, 128), jnp.float32),
                   jax.ShapeDtypeStruct((8, 128), jnp.float32)],
    )(agg, h2n, dinv2, b_gcn2.reshape(1, 128))


def _p7_body(h2_ref, hn_ref, s_ref, t_ref, o_ref):
    o_ref[...] = hn_ref[...] + jax.nn.relu(h2_ref[...] * s_ref[...] + t_ref[...])


def tc_p7(h2, h1n, s, t, blk):
    m = h2.shape[0]
    return pl.pallas_call(
        _p7_body,
        grid=(m // blk,),
        in_specs=[_rows_spec(blk, 128), _rows_spec(blk, 128),
                  pl.BlockSpec((1, 128), lambda i: (0, 0)),
                  pl.BlockSpec((1, 128), lambda i: (0, 0))],
        out_specs=_rows_spec(blk, 128),
        out_shape=jax.ShapeDtypeStruct((m, 128), jnp.float32),
    )(h2, h1n, s.reshape(1, 128), t.reshape(1, 128))


def _ln(x, g, b):
    mu = jnp.mean(x, axis=-1, keepdims=True)
    var = jnp.mean((x - mu) ** 2, axis=-1, keepdims=True)
    return (x - mu) / jnp.sqrt(var + EPS) * g + b


def _p8_body(sums_ref, cnt_ref, w0_ref, b0_ref, g0_ref, t0_ref,
             w1_ref, b1_ref, g1_ref, t1_ref, w2_ref, b2_ref, o_ref):
    hp = sums_ref[...] / jnp.maximum(cnt_ref[...], 1.0)
    h = _gelu(_ln(jnp.dot(hp, w0_ref[...], preferred_element_type=jnp.float32)
                  + b0_ref[...], g0_ref[...], t0_ref[...]))
    h = _gelu(_ln(jnp.dot(h, w1_ref[...], preferred_element_type=jnp.float32)
                  + b1_ref[...], g1_ref[...], t1_ref[...]))
    o_ref[...] = jnp.dot(h, w2_ref[...], preferred_element_type=jnp.float32) \
        + b2_ref[...]


def tc_p8(sums, cnt, w_f0, b_f0, g_ln0, t_ln0, w_f1, b_f1, g_ln1, t_ln1,
          w_f2, b_f2):
    full = lambda shape: pl.BlockSpec(shape, lambda: (0,) * len(shape))
    return pl.pallas_call(
        _p8_body,
        in_specs=[full((256, 128)), full((256, 1)),
                  full((128, 128)), full((1, 128)), full((1, 128)), full((1, 128)),
                  full((128, 128)), full((1, 128)), full((1, 128)), full((1, 128)),
                  full((128, 1)), full((1, 1))],
        out_specs=full((256, 1)),
        out_shape=jax.ShapeDtypeStruct((256, 1), jnp.float32),
    )(sums, cnt.reshape(256, 1), w_f0, b_f0.reshape(1, 128),
      g_ln0.reshape(1, 128), t_ln0.reshape(1, 128), w_f1, b_f1.reshape(1, 128),
      g_ln1.reshape(1, 128), t_ln1.reshape(1, 128), w_f2, b_f2.reshape(1, 1))


def _bn_affine(mu, m2, n, g, b):
    var = m2 / n
    s = g / jnp.sqrt(var + EPS)
    return s, b - mu * s


def kernel(x, edge_index, batch, undirected_edge_mask, l2_node_mapping,
           l2_edge_index, num_graphs, W_gcn1, b_gcn1, W_ne0, b_ne0, g_ne,
           bt_ne, W_ne1, b_ne1, W_ef0, b_ef0, g_ef, bt_ef, W_ef1, b_ef1,
           g_bn1, bt_bn1, g_bn2, bt_bn2, W_gcn2, b_gcn2, W_f0, b_f0, g_ln0,
           bt_ln0, W_f1, b_f1, g_ln1, bt_ln1, W_f2, b_f2):
    n = x.shape[0]
    e = edge_index.shape[1]
    m = l2_node_mapping.shape[1]
    src, dst = edge_index[0], edge_index[1]

    pad1 = 163840 - e
    src1p = jnp.concatenate([src, jnp.zeros((pad1,), src.dtype)])
    dst1p = jnp.concatenate([dst, jnp.full((pad1,), n, dst.dtype)])
    deg1 = jnp.zeros((n,), jnp.float32).at[dst].add(1.0) + 1.0
    dinv1 = jax.lax.rsqrt(deg1).reshape(n, 1)
    h1n = tc_scale_rows_matmul(x, W_gcn1, dinv1, 2000)
    agg1 = sc_gcn_agg1(h1n, src1p, dst1p)
    H0 = tc_gcn_post(agg1, h1n, dinv1, b_gcn1, 2000)

    Hu = H0[src]
    Hv = H0[dst]
    U, Esum, st1 = tc_p1(Hu, Hv, W_ne0, b_ne0, 4000)
    g2 = jnp.concatenate([g_ne, g_ne])
    b2 = jnp.concatenate([bt_ne, bt_ne])
    s_ne, t_ne = _bn_affine(st1[0], st1[1], float(e), g2, b2)
    h_edge = tc_p2(U, s_ne, t_ne, W_ne1, b_ne1, 4000)

    e_idx = l2_node_mapping[0]
    f_idx = l2_node_mapping[1]
    he = h_edge[e_idx]
    hf = h_edge[f_idx]
    ee = Esum[e_idx]
    ef = Esum[f_idx]
    V, H0m, st3 = tc_p3(he, hf, ee, ef, W_ef0, b_ef0, 4000)
    s_ef1, t_ef1 = _bn_affine(st3[0], st3[1], float(m), g_ef, bt_ef)
    s_ef2, t_ef2 = _bn_affine(st3[2], st3[3], float(m), g_ef, bt_ef)
    s_ef = jnp.concatenate([s_ef1, s_ef2])
    t_ef = jnp.concatenate([t_ef1, t_ef2])
    H1, st4 = tc_p4(V, s_ef, t_ef, W_ef1, b_ef1, 4000)
    s_b1, t_b1 = _bn_affine(st4[0], st4[1], float(m), g_bn1, bt_bn1)

    src2, dst2 = l2_edge_index[0], l2_edge_index[1]
    e2 = src2.shape[0]
    pad2 = 409600 - e2
    src2p = jnp.concatenate([src2, jnp.zeros((pad2,), src2.dtype)])
    dst2p = jnp.concatenate([dst2, jnp.full((pad2,), m, dst2.dtype)])
    deg2 = jnp.zeros((m,), jnp.float32).at[dst2].add(1.0) + 1.0
    dinv2 = jax.lax.rsqrt(deg2).reshape(m, 1)
    H1_new, h2n = tc_p5(H1, H0m, s_b1, t_b1, dinv2, W_gcn2, 4000)
    agg2 = sc_gcn_agg2(h2n, src2p, dst2p)
    H2, st6 = tc_p6(agg2, h2n, dinv2, b_gcn2, 4000)
    s_b2, t_b2 = _bn_affine(st6[0], st6[1], float(m), g_bn2, bt_bn2)
    H2_new = tc_p7(H2, H1_new, s_b2, t_b2, 4000)

    l2_batch = batch[src[e_idx]].astype(jnp.int32)
    sums_p, cnt_p = sc_segsum(H2_new, l2_batch)
    sums = sums_p[0, :256] + sums_p[1, :256]
    cnt = cnt_p[0, :256, 0] + cnt_p[1, :256, 0]
    return tc_p8(sums, cnt, W_f0, b_f0, g_ln0, bt_ln0, W_f1, b_f1,
                 g_ln1, bt_ln1, W_f2, b_f2)


# speedup vs baseline: 1.4073x; 1.4073x over previous
import functools

import jax
import jax.numpy as jnp
from jax import lax
from jax.experimental import pallas as pl
from jax.experimental.pallas import tpu as pltpu
from jax.experimental.pallas import tpu_sc as plsc

EPS = 1e-5

_NS = 16
_CH = 512
_IDXU = _CH // 16


def _sc_mesh():
    return plsc.VectorSubcoreMesh(core_axis_name="c", subcore_axis_name="s")


def _zero_rows(buf, nrows, ncol16):
    z = jnp.zeros((16,), jnp.float32)

    @pl.loop(0, nrows)
    def _(j):
        for c in range(ncol16):
            buf[j, pl.ds(c * 16, 16)] = z


def _fill_ones(buf, nrows, ncol16):
    o = jnp.ones((16,), jnp.float32)

    @pl.loop(0, nrows)
    def _(j):
        for c in range(ncol16):
            buf[j, pl.ds(c * 16, 16)] = o


def _sc_gcn_agg_maker(half, colw, colgroups, e_pad, with_gather,
                      wb_chunk, wb_nch):
    half_pad = _NS * wb_chunk * wb_nch
    per_tile = e_pad // _NS
    n_ch = per_tile // _CH
    zch = -(-max(half + 8, half_pad) // (_NS * _CH))
    acc_rows = zch * _NS * _CH
    dump = half

    scratch = [
        pltpu.VMEM((_CH,), jnp.int32),
        pltpu.VMEM((_CH,), jnp.int32),
        pltpu.VMEM((_CH,), jnp.int32),
        pltpu.VMEM((_CH,), jnp.int32),
        pltpu.VMEM((_CH, colw), jnp.float32),
        pltpu.VMEM((wb_chunk, colw), jnp.float32),
        pltpu.VMEM_SHARED((acc_rows, colw), jnp.float32),
        pltpu.SemaphoreType.DMA,
    ]

    def body(*refs):
        if with_gather:
            (tbl, srch, dsth, out, srcv, dstv, gidx, ldx, rows, stag, acc,
             sem) = refs
        else:
            (dsth, out, srcv, dstv, gidx, ldx, rows, stag, acc, sem) = refs
            tbl = None
        cid = lax.axis_index("c")
        sid = lax.axis_index("s")
        base = cid * half
        ebase = sid * per_tile

        @pl.loop(0, colgroups)
        def _cg(cg):
            _zero_rows(rows, _CH, colw // 16)

            @pl.loop(0, zch)
            def _(k):
                pltpu.sync_copy(rows, acc.at[pl.ds(sid * zch * _CH
                                                   + k * _CH, _CH)])
            plsc.subcore_barrier()
            if not with_gather:
                _fill_ones(rows, _CH, colw // 16)

            @pl.loop(0, n_ch)
            def _(ch):
                off = ebase + ch * _CH
                if with_gather:
                    pltpu.sync_copy(srch.at[pl.ds(off, _CH)], srcv)
                pltpu.sync_copy(dsth.at[pl.ds(off, _CH)], dstv)
                for k in range(_IDXU):
                    d16 = dstv[pl.ds(k * 16, 16)]
                    ok = (d16 >= base) & (d16 < base + half)
                    l16 = jnp.where(ok, d16 - base, dump)
                    ldx[pl.ds(k * 16, 16)] = l16
                    if with_gather:
                        s16 = srcv[pl.ds(k * 16, 16)]
                        g16 = s16 * colgroups + cg
                        gidx[pl.ds(k * 16, 16)] = g16
                if with_gather:
                    pltpu.async_copy(tbl.at[gidx], rows, sem).wait()
                pltpu.sync_copy(rows, acc.at[ldx], add=True)
            plsc.subcore_barrier()

            @pl.loop(0, wb_nch)
            def _(w):
                r0 = sid * wb_nch * wb_chunk + w * wb_chunk
                pltpu.sync_copy(acc.at[pl.ds(r0, wb_chunk)], stag)
                pltpu.sync_copy(stag, out.at[cg, pl.ds(cid * half_pad + r0,
                                                       wb_chunk)])
            plsc.subcore_barrier()

    kern = pl.kernel(
        body,
        out_type=jax.ShapeDtypeStruct((colgroups, 2 * half_pad, colw),
                                      jnp.float32),
        mesh=_sc_mesh(),
        scratch_types=scratch,
        compiler_params=pltpu.CompilerParams(use_tc_tiling_on_sc=False),
    )
    return kern


def sc_gcn_agg1(h1n, src_p, dst_p):
    k = _sc_gcn_agg_maker(5000, 64, 2, src_p.shape[0], True, 160, 2)
    o = k(h1n.reshape(-1, 64), src_p, dst_p)
    o = jnp.concatenate([o[:, :5000], o[:, 5120:10120]], axis=1)
    return jnp.concatenate([o[0], o[1]], axis=1)


def sc_gcn_agg2(h2n, src_p, dst_p):
    k = _sc_gcn_agg_maker(100000, 16, 8, src_p.shape[0], True, 640, 10)
    o = k(h2n.reshape(-1, 16), src_p, dst_p)
    return jnp.concatenate([o[:, :100000], o[:, 102400:202400]], axis=1)


def sc_degree(dst_p, half, wb_chunk, wb_nch):
    k = _sc_gcn_agg_maker(half, 16, 1, dst_p.shape[0], False,
                          wb_chunk, wb_nch)
    o = k(dst_p)[0]
    hp = _NS * wb_chunk * wb_nch
    return jnp.concatenate([o[:half], o[hp:hp + half]], axis=0)


def sc_segsum(h2new, l2b2):
    m = h2new.shape[0]
    chr_ = 400
    per_tile = 8000
    n_ch = per_tile // chr_

    def body(vals_hbm, idx_hbm, sums_out, cnt_out, vbuf, obuf, ibuf,
             acc_s, acc_c, sem):
        cid = lax.axis_index("c")
        sid = lax.axis_index("s")
        tg = cid * _NS + sid
        _zero_rows(vbuf, 264, 8)
        _zero_rows(obuf, 264, 1)

        @pl.when(sid == 0)
        def _():
            pltpu.sync_copy(vbuf.at[pl.ds(0, 264)], acc_s)
            pltpu.sync_copy(obuf.at[pl.ds(0, 264)], acc_c)
        plsc.subcore_barrier()
        _fill_ones(obuf, chr_, 1)

        @pl.when(tg < 25)
        def _():
            @pl.loop(0, n_ch)
            def _(ch):
                off = tg * per_tile + ch * chr_
                pltpu.sync_copy(vals_hbm.at[pl.ds(off, chr_)], vbuf)
                pltpu.sync_copy(idx_hbm.at[pl.ds(off, chr_)], ibuf)
                pltpu.sync_copy(vbuf, acc_s.at[ibuf], add=True)
                pltpu.sync_copy(obuf, acc_c.at[ibuf], add=True)
        plsc.subcore_barrier()

        @pl.when(sid == 0)
        def _():
            pltpu.sync_copy(acc_s, sums_out.at[cid])
            pltpu.sync_copy(acc_c, cnt_out.at[cid])

    kern = pl.kernel(
        body,
        out_type=[jax.ShapeDtypeStruct((2, 264, 128), jnp.float32),
                  jax.ShapeDtypeStruct((2, 264, 16), jnp.float32)],
        mesh=_sc_mesh(),
        scratch_types=[
            pltpu.VMEM((chr_, 128), jnp.float32),
            pltpu.VMEM((chr_, 16), jnp.float32),
            pltpu.VMEM((chr_,), jnp.int32),
            pltpu.VMEM_SHARED((264, 128), jnp.float32),
            pltpu.VMEM_SHARED((264, 16), jnp.float32),
            pltpu.SemaphoreType.DMA,
        ],
        compiler_params=pltpu.CompilerParams(use_tc_tiling_on_sc=False),
    )
    return kern(h2new, l2b2)


def _gelu(x):
    return 0.5 * x * (1.0 + jax.lax.erf(x * 0.7071067811865476))


def _rows_spec(blk, w):
    return pl.BlockSpec((blk, w), lambda i: (i, 0))


def _stats_spec():
    return pl.BlockSpec((8, 128), lambda i: (0, 0))


def _stats_merge(st_ref, r, x, blk):
    i = pl.program_id(0)
    mb = jnp.sum(x, axis=0, keepdims=True) * (1.0 / blk)
    m2b = jnp.sum((x - mb) ** 2, axis=0, keepdims=True)

    @pl.when(i == 0)
    def _():
        st_ref[r:r + 1] = mb
        st_ref[r + 1:r + 2] = m2b

    @pl.when(i != 0)
    def _():
        nf = i.astype(jnp.float32) * float(blk)
        mean = st_ref[r:r + 1]
        delta = mb - mean
        tot = nf + float(blk)
        st_ref[r:r + 1] = mean + delta * (float(blk) / tot)
        st_ref[r + 1:r + 2] = (st_ref[r + 1:r + 2] + m2b
                               + delta * delta * (nf * float(blk) / tot))


def _scale_rows_matmul(x_ref, w_ref, s_ref, o_ref):
    h = jnp.dot(x_ref[...], w_ref[...], preferred_element_type=jnp.float32)
    o_ref[...] = s_ref[...] * h


def tc_scale_rows_matmul(x, w, s, blk):
    n, d = x.shape
    return pl.pallas_call(
        _scale_rows_matmul,
        grid=(n // blk,),
        in_specs=[_rows_spec(blk, d), pl.BlockSpec((d, w.shape[1]), lambda i: (0, 0)),
                  _rows_spec(blk, 1)],
        out_specs=_rows_spec(blk, w.shape[1]),
        out_shape=jax.ShapeDtypeStruct((n, w.shape[1]), jnp.float32),
    )(x, w, s)


def _gcn_post(agg_ref, hn_ref, s_ref, b_ref, o_ref):
    o_ref[...] = s_ref[...] * (agg_ref[...] + hn_ref[...]) + b_ref[...]


def tc_gcn_post(agg, hn, s, b, blk):
    n, d = agg.shape
    return pl.pallas_call(
        _gcn_post,
        grid=(n // blk,),
        in_specs=[_rows_spec(blk, d), _rows_spec(blk, d), _rows_spec(blk, 1),
                  pl.BlockSpec((1, d), lambda i: (0, 0))],
        out_specs=_rows_spec(blk, d),
        out_shape=jax.ShapeDtypeStruct((n, d), jnp.float32),
    )(agg, hn, s, b.reshape(1, d))


def _p1_body(blk, hu_ref, hv_ref, w_ref, b_ref, u_ref, es_ref, st_ref):
    hu = hu_ref[...]
    hv = hv_ref[...]
    w = w_ref[...]
    b = b_ref[...]
    c1 = jnp.concatenate([hu, hv], axis=1)
    c2 = jnp.concatenate([hv, hu], axis=1)
    u1 = jnp.dot(c1, w, preferred_element_type=jnp.float32) + b
    u2 = jnp.dot(c2, w, preferred_element_type=jnp.float32) + b
    u = jnp.concatenate([u1, u2], axis=1)
    u_ref[...] = u
    es_ref[...] = hu + hv
    _stats_merge(st_ref, 0, u, blk)


def tc_p1(hu, hv, w_ne0, b_ne0, blk):
    e = hu.shape[0]
    return pl.pallas_call(
        functools.partial(_p1_body, float(blk)),
        grid=(e // blk,),
        in_specs=[_rows_spec(blk, 128), _rows_spec(blk, 128),
                  pl.BlockSpec((256, 64), lambda i: (0, 0)),
                  pl.BlockSpec((1, 64), lambda i: (0, 0))],
        out_specs=[_rows_spec(blk, 128), _rows_spec(blk, 128), _stats_spec()],
        out_shape=[jax.ShapeDtypeStruct((e, 128), jnp.float32),
                   jax.ShapeDtypeStruct((e, 128), jnp.float32),
                   jax.ShapeDtypeStruct((8, 128), jnp.float32)],
    )(hu, hv, w_ne0, b_ne0.reshape(1, 64))


def _p2_body(u_ref, s_ref, t_ref, w_ref, b_ref, o_ref):
    g = _gelu(u_ref[...] * s_ref[...] + t_ref[...])
    w = w_ref[...]
    b = b_ref[...]
    o1 = jnp.dot(g[:, :64], w, preferred_element_type=jnp.float32) + b
    o2 = jnp.dot(g[:, 64:], w, preferred_element_type=jnp.float32) + b
    o_ref[...] = 0.5 * (o1 + o2)


def tc_p2(u, s, t, w_ne1, b_ne1, blk):
    e = u.shape[0]
    return pl.pallas_call(
        _p2_body,
        grid=(e // blk,),
        in_specs=[_rows_spec(blk, 128),
                  pl.BlockSpec((1, 128), lambda i: (0, 0)),
                  pl.BlockSpec((1, 128), lambda i: (0, 0)),
                  pl.BlockSpec((64, 64), lambda i: (0, 0)),
                  pl.BlockSpec((1, 64), lambda i: (0, 0))],
        out_specs=_rows_spec(blk, 64),
        out_shape=jax.ShapeDtypeStruct((e, 64), jnp.float32),
    )(u, s.reshape(1, 128), t.reshape(1, 128), w_ne1, b_ne1.reshape(1, 64))


def _p3_body(blk, he_ref, hf_ref, ee_ref, ef_ref, w_ref, b_ref,
             v_ref, hm_ref, st_ref):
    he = he_ref[...]
    hf = hf_ref[...]
    w = w_ref[...]
    b = b_ref[...]
    v1 = jnp.dot(jnp.concatenate([hf, he], axis=1), w,
                 preferred_element_type=jnp.float32) + b
    v2 = jnp.dot(jnp.concatenate([he, hf], axis=1), w,
                 preferred_element_type=jnp.float32) + b
    v_ref[:, :128] = v1
    v_ref[:, 128:] = v2
    hm_ref[...] = 0.25 * (ee_ref[...] + ef_ref[...])
    _stats_merge(st_ref, 0, v1, blk)
    _stats_merge(st_ref, 2, v2, blk)


def tc_p3(he, hf, ee, ef, w_ef0, b_ef0, blk):
    m = he.shape[0]
    return pl.pallas_call(
        functools.partial(_p3_body, float(blk)),
        grid=(m // blk,),
        in_specs=[_rows_spec(blk, 64), _rows_spec(blk, 64),
                  _rows_spec(blk, 128), _rows_spec(blk, 128),
                  pl.BlockSpec((128, 128), lambda i: (0, 0)),
                  pl.BlockSpec((1, 128), lambda i: (0, 0))],
        out_specs=[_rows_spec(blk, 256), _rows_spec(blk, 128), _stats_spec()],
        out_shape=[jax.ShapeDtypeStruct((m, 256), jnp.float32),
                   jax.ShapeDtypeStruct((m, 128), jnp.float32),
                   jax.ShapeDtypeStruct((8, 128), jnp.float32)],
    )(he, hf, ee, ef, w_ef0, b_ef0.reshape(1, 128))


def _p4_body(blk, v_ref, s_ref, t_ref, w_ref, b_ref, h1_ref, st_ref):
    s = s_ref[...]
    t = t_ref[...]
    g1 = _gelu(v_ref[:, :128] * s[:, :128] + t[:, :128])
    g2 = _gelu(v_ref[:, 128:] * s[:, 128:] + t[:, 128:])
    w = w_ref[...]
    b = b_ref[...]
    h1a = jnp.dot(g1, w, preferred_element_type=jnp.float32) + b
    h1b = jnp.dot(g2, w, preferred_element_type=jnp.float32) + b
    h1 = 0.5 * (h1a + h1b)
    h1_ref[...] = h1
    _stats_merge(st_ref, 0, h1, blk)


def tc_p4(v, s, t, w_ef1, b_ef1, blk):
    m = v.shape[0]
    return pl.pallas_call(
        functools.partial(_p4_body, float(blk)),
        grid=(m // blk,),
        in_specs=[_rows_spec(blk, 256),
                  pl.BlockSpec((1, 256), lambda i: (0, 0)),
                  pl.BlockSpec((1, 256), lambda i: (0, 0)),
                  pl.BlockSpec((128, 128), lambda i: (0, 0)),
                  pl.BlockSpec((1, 128), lambda i: (0, 0))],
        out_specs=[_rows_spec(blk, 128), _stats_spec()],
        out_shape=[jax.ShapeDtypeStruct((m, 128), jnp.float32),
                   jax.ShapeDtypeStruct((8, 128), jnp.float32)],
    )(v, s.reshape(1, 256), t.reshape(1, 256), w_ef1, b_ef1.reshape(1, 128))


def _p5_body(h1_ref, hm_ref, s_ref, t_ref, dinv_ref, w_ref, hn_ref, h2n_ref):
    h1n = hm_ref[...] + jax.nn.relu(h1_ref[...] * s_ref[...] + t_ref[...])
    hn_ref[...] = h1n
    h2 = jnp.dot(h1n, w_ref[...], preferred_element_type=jnp.float32)
    h2n_ref[...] = dinv_ref[...] * h2


def tc_p5(h1, hm, s, t, dinv2, w_gcn2, blk):
    m = h1.shape[0]
    return pl.pallas_call(
        _p5_body,
        grid=(m // blk,),
        in_specs=[_rows_spec(blk, 128), _rows_spec(blk, 128),
                  pl.BlockSpec((1, 128), lambda i: (0, 0)),
                  pl.BlockSpec((1, 128), lambda i: (0, 0)),
                  _rows_spec(blk, 1),
                  pl.BlockSpec((128, 128), lambda i: (0, 0))],
        out_specs=[_rows_spec(blk, 128), _rows_spec(blk, 128)],
        out_shape=[jax.ShapeDtypeStruct((m, 128), jnp.float32),
                   jax.ShapeDtypeStruct((m, 128), jnp.float32)],
    )(h1, hm, s.reshape(1, 128), t.reshape(1, 128), dinv2, w_gcn2)


def _p6_body(blk, agg_ref, h2n_ref, dinv_ref, b_ref, h2_ref, st_ref):
    h2 = dinv_ref[...] * (agg_ref[...] + h2n_ref[...]) + b_ref[...]
    h2_ref[...] = h2
    _stats_merge(st_ref, 0, h2, blk)


def tc_p6(agg, h2n, dinv2, b_gcn2, blk):
    m = h2n.shape[0]
    return pl.pallas_call(
        functools.partial(_p6_body, float(blk)),
        grid=(m // blk,),
        in_specs=[_rows_spec(blk, 128),
                  _rows_spec(blk, 128), _rows_spec(blk, 1),
                  pl.BlockSpec((1, 128), lambda i: (0, 0))],
        out_specs=[_rows_spec(blk, 128), _stats_spec()],
        out_shape=[jax.ShapeDtypeStruct((m, 128), jnp.float32),
                   jax.ShapeDtypeStruct((8, 128), jnp.float32)],
    )(agg, h2n, dinv2, b_gcn2.reshape(1, 128))


def _p7_body(h2_ref, hn_ref, s_ref, t_ref, o_ref):
    o_ref[...] = hn_ref[...] + jax.nn.relu(h2_ref[...] * s_ref[...] + t_ref[...])


def tc_p7(h2, h1n, s, t, blk):
    m = h2.shape[0]
    return pl.pallas_call(
        _p7_body,
        grid=(m // blk,),
        in_specs=[_rows_spec(blk, 128), _rows_spec(blk, 128),
                  pl.BlockSpec((1, 128), lambda i: (0, 0)),
                  pl.BlockSpec((1, 128), lambda i: (0, 0))],
        out_specs=_rows_spec(blk, 128),
        out_shape=jax.ShapeDtypeStruct((m, 128), jnp.float32),
    )(h2, h1n, s.reshape(1, 128), t.reshape(1, 128))


def _ln(x, g, b):
    mu = jnp.mean(x, axis=-1, keepdims=True)
    var = jnp.mean((x - mu) ** 2, axis=-1, keepdims=True)
    return (x - mu) / jnp.sqrt(var + EPS) * g + b


def _p8_body(sums_ref, cnt_ref, w0_ref, b0_ref, g0_ref, t0_ref,
             w1_ref, b1_ref, g1_ref, t1_ref, w2_ref, b2_ref, o_ref):
    hp = sums_ref[...] / jnp.maximum(cnt_ref[...], 1.0)
    h = _gelu(_ln(jnp.dot(hp, w0_ref[...], preferred_element_type=jnp.float32)
                  + b0_ref[...], g0_ref[...], t0_ref[...]))
    h = _gelu(_ln(jnp.dot(h, w1_ref[...], preferred_element_type=jnp.float32)
                  + b1_ref[...], g1_ref[...], t1_ref[...]))
    o_ref[...] = jnp.dot(h, w2_ref[...], preferred_element_type=jnp.float32) \
        + b2_ref[...]


def tc_p8(sums, cnt, w_f0, b_f0, g_ln0, t_ln0, w_f1, b_f1, g_ln1, t_ln1,
          w_f2, b_f2):
    full = lambda shape: pl.BlockSpec(shape, lambda: (0,) * len(shape))
    return pl.pallas_call(
        _p8_body,
        in_specs=[full((256, 128)), full((256, 1)),
                  full((128, 128)), full((1, 128)), full((1, 128)), full((1, 128)),
                  full((128, 128)), full((1, 128)), full((1, 128)), full((1, 128)),
                  full((128, 1)), full((1, 1))],
        out_specs=full((256, 1)),
        out_shape=jax.ShapeDtypeStruct((256, 1), jnp.float32),
    )(sums, cnt.reshape(256, 1), w_f0, b_f0.reshape(1, 128),
      g_ln0.reshape(1, 128), t_ln0.reshape(1, 128), w_f1, b_f1.reshape(1, 128),
      g_ln1.reshape(1, 128), t_ln1.reshape(1, 128), w_f2, b_f2.reshape(1, 1))


def _bn_affine(mu, m2, n, g, b):
    var = m2 / n
    s = g / jnp.sqrt(var + EPS)
    return s, b - mu * s


def kernel(x, edge_index, batch, undirected_edge_mask, l2_node_mapping,
           l2_edge_index, num_graphs, W_gcn1, b_gcn1, W_ne0, b_ne0, g_ne,
           bt_ne, W_ne1, b_ne1, W_ef0, b_ef0, g_ef, bt_ef, W_ef1, b_ef1,
           g_bn1, bt_bn1, g_bn2, bt_bn2, W_gcn2, b_gcn2, W_f0, b_f0, g_ln0,
           bt_ln0, W_f1, b_f1, g_ln1, bt_ln1, W_f2, b_f2):
    n = x.shape[0]
    e = edge_index.shape[1]
    m = l2_node_mapping.shape[1]
    src, dst = edge_index[0], edge_index[1]

    pad1 = 163840 - e
    src1p = jnp.concatenate([src, jnp.zeros((pad1,), src.dtype)])
    dst1p = jnp.concatenate([dst, jnp.full((pad1,), n, dst.dtype)])
    deg1 = jnp.zeros((n,), jnp.float32).at[dst].add(1.0) + 1.0
    dinv1 = jax.lax.rsqrt(deg1).reshape(n, 1)
    h1n = tc_scale_rows_matmul(x, W_gcn1, dinv1, 2000)
    agg1 = sc_gcn_agg1(h1n, src1p, dst1p)
    H0 = tc_gcn_post(agg1, h1n, dinv1, b_gcn1, 2000)

    Hu = H0[src]
    Hv = H0[dst]
    U, Esum, st1 = tc_p1(Hu, Hv, W_ne0, b_ne0, 4000)
    g2 = jnp.concatenate([g_ne, g_ne])
    b2 = jnp.concatenate([bt_ne, bt_ne])
    s_ne, t_ne = _bn_affine(st1[0], st1[1], float(e), g2, b2)
    h_edge = tc_p2(U, s_ne, t_ne, W_ne1, b_ne1, 4000)

    e_idx = l2_node_mapping[0]
    f_idx = l2_node_mapping[1]
    he = h_edge[e_idx]
    hf = h_edge[f_idx]
    ee = Esum[e_idx]
    ef = Esum[f_idx]
    V, H0m, st3 = tc_p3(he, hf, ee, ef, W_ef0, b_ef0, 4000)
    s_ef1, t_ef1 = _bn_affine(st3[0], st3[1], float(m), g_ef, bt_ef)
    s_ef2, t_ef2 = _bn_affine(st3[2], st3[3], float(m), g_ef, bt_ef)
    s_ef = jnp.concatenate([s_ef1, s_ef2])
    t_ef = jnp.concatenate([t_ef1, t_ef2])
    H1, st4 = tc_p4(V, s_ef, t_ef, W_ef1, b_ef1, 4000)
    s_b1, t_b1 = _bn_affine(st4[0], st4[1], float(m), g_bn1, bt_bn1)

    src2, dst2 = l2_edge_index[0], l2_edge_index[1]
    e2 = src2.shape[0]
    pad2 = 409600 - e2
    src2p = jnp.concatenate([src2, jnp.zeros((pad2,), src2.dtype)])
    dst2p = jnp.concatenate([dst2, jnp.full((pad2,), m, dst2.dtype)])
    deg2 = jnp.zeros((m,), jnp.float32).at[dst2].add(1.0) + 1.0
    dinv2 = jax.lax.rsqrt(deg2).reshape(m, 1)
    H1_new, h2n = tc_p5(H1, H0m, s_b1, t_b1, dinv2, W_gcn2, 4000)
    agg2 = jnp.zeros((m, 128), jnp.float32).at[dst2].add(h2n[src2])
    H2, st6 = tc_p6(agg2, h2n, dinv2, b_gcn2, 4000)
    s_b2, t_b2 = _bn_affine(st6[0], st6[1], float(m), g_bn2, bt_bn2)
    H2_new = tc_p7(H2, H1_new, s_b2, t_b2, 4000)

    l2_batch = batch[src[e_idx]].astype(jnp.int32)
    sums_p, cnt_p = sc_segsum(H2_new, l2_batch)
    sums = sums_p[0, :256] + sums_p[1, :256]
    cnt = cnt_p[0, :256, 0] + cnt_p[1, :256, 0]
    return tc_p8(sums, cnt, W_f0, b_f0, g_ln0, bt_ln0, W_f1, b_f1,
                 g_ln1, bt_ln1, W_f2, b_f2)


# speedup vs baseline: 1.4074x; 1.0000x over previous
import functools

import jax
import jax.numpy as jnp
from jax import lax
from jax.experimental import pallas as pl
from jax.experimental.pallas import tpu as pltpu
from jax.experimental.pallas import tpu_sc as plsc

EPS = 1e-5

_NS = 16
_CH = 512
_IDXU = _CH // 16


def _sc_mesh():
    return plsc.VectorSubcoreMesh(core_axis_name="c", subcore_axis_name="s")


def _zero_rows(buf, nrows, ncol16):
    z = jnp.zeros((16,), jnp.float32)

    @pl.loop(0, nrows)
    def _(j):
        for c in range(ncol16):
            buf[j, pl.ds(c * 16, 16)] = z


def _fill_ones(buf, nrows, ncol16):
    o = jnp.ones((16,), jnp.float32)

    @pl.loop(0, nrows)
    def _(j):
        for c in range(ncol16):
            buf[j, pl.ds(c * 16, 16)] = o


def _sc_gcn_agg_maker(half, colw, colgroups, e_pad, with_gather,
                      wb_chunk, wb_nch):
    half_pad = _NS * wb_chunk * wb_nch
    per_tile = e_pad // _NS
    n_ch = per_tile // _CH
    zch = -(-max(half + 8, half_pad) // (_NS * _CH))
    acc_rows = zch * _NS * _CH
    dump = half

    scratch = [
        pltpu.VMEM((_CH,), jnp.int32),
        pltpu.VMEM((_CH,), jnp.int32),
        pltpu.VMEM((_CH,), jnp.int32),
        pltpu.VMEM((_CH,), jnp.int32),
        pltpu.VMEM((_CH, colw), jnp.float32),
        pltpu.VMEM((wb_chunk, colw), jnp.float32),
        pltpu.VMEM_SHARED((acc_rows, colw), jnp.float32),
        pltpu.SemaphoreType.DMA,
    ]

    def body(*refs):
        if with_gather:
            (tbl, srch, dsth, out, srcv, dstv, gidx, ldx, rows, stag, acc,
             sem) = refs
        else:
            (dsth, out, srcv, dstv, gidx, ldx, rows, stag, acc, sem) = refs
            tbl = None
        cid = lax.axis_index("c")
        sid = lax.axis_index("s")
        base = cid * half
        ebase = sid * per_tile

        @pl.loop(0, colgroups)
        def _cg(cg):
            _zero_rows(rows, _CH, colw // 16)

            @pl.loop(0, zch)
            def _(k):
                pltpu.sync_copy(rows, acc.at[pl.ds(sid * zch * _CH
                                                   + k * _CH, _CH)])
            plsc.subcore_barrier()
            if not with_gather:
                _fill_ones(rows, _CH, colw // 16)

            @pl.loop(0, n_ch)
            def _(ch):
                off = ebase + ch * _CH
                if with_gather:
                    pltpu.sync_copy(srch.at[pl.ds(off, _CH)], srcv)
                pltpu.sync_copy(dsth.at[pl.ds(off, _CH)], dstv)
                for k in range(_IDXU):
                    d16 = dstv[pl.ds(k * 16, 16)]
                    ok = (d16 >= base) & (d16 < base + half)
                    l16 = jnp.where(ok, d16 - base, dump)
                    ldx[pl.ds(k * 16, 16)] = l16
                    if with_gather:
                        s16 = srcv[pl.ds(k * 16, 16)]
                        g16 = s16 * colgroups + cg
                        gidx[pl.ds(k * 16, 16)] = g16
                if with_gather:
                    pltpu.async_copy(tbl.at[gidx], rows, sem).wait()
                pltpu.sync_copy(rows, acc.at[ldx], add=True)
            plsc.subcore_barrier()

            @pl.loop(0, wb_nch)
            def _(w):
                r0 = sid * wb_nch * wb_chunk + w * wb_chunk
                pltpu.sync_copy(acc.at[pl.ds(r0, wb_chunk)], stag)
                pltpu.sync_copy(stag, out.at[cg, pl.ds(cid * half_pad + r0,
                                                       wb_chunk)])
            plsc.subcore_barrier()

    kern = pl.kernel(
        body,
        out_type=jax.ShapeDtypeStruct((colgroups, 2 * half_pad, colw),
                                      jnp.float32),
        mesh=_sc_mesh(),
        scratch_types=scratch,
        compiler_params=pltpu.CompilerParams(use_tc_tiling_on_sc=False),
    )
    return kern


def sc_gcn_agg1(h1n, src_p, dst_p):
    k = _sc_gcn_agg_maker(5000, 64, 2, src_p.shape[0], True, 160, 2)
    o = k(h1n.reshape(-1, 64), src_p, dst_p)
    o = jnp.concatenate([o[:, :5000], o[:, 5120:10120]], axis=1)
    return jnp.concatenate([o[0], o[1]], axis=1)


def sc_segsum(h2new, l2b2):
    m = h2new.shape[0]
    chr_ = 400
    per_tile = 8000
    n_ch = per_tile // chr_

    def body(vals_hbm, idx_hbm, sums_out, cnt_out, vbuf, obuf, ibuf,
             acc_s, acc_c, sem):
        cid = lax.axis_index("c")
        sid = lax.axis_index("s")
        tg = cid * _NS + sid
        _zero_rows(vbuf, 264, 8)
        _zero_rows(obuf, 264, 1)

        @pl.when(sid == 0)
        def _():
            pltpu.sync_copy(vbuf.at[pl.ds(0, 264)], acc_s)
            pltpu.sync_copy(obuf.at[pl.ds(0, 264)], acc_c)
        plsc.subcore_barrier()
        _fill_ones(obuf, chr_, 1)

        @pl.when(tg < 25)
        def _():
            @pl.loop(0, n_ch)
            def _(ch):
                off = tg * per_tile + ch * chr_
                pltpu.sync_copy(vals_hbm.at[pl.ds(off, chr_)], vbuf)
                pltpu.sync_copy(idx_hbm.at[pl.ds(off, chr_)], ibuf)
                pltpu.sync_copy(vbuf, acc_s.at[ibuf], add=True)
                pltpu.sync_copy(obuf, acc_c.at[ibuf], add=True)
        plsc.subcore_barrier()

        @pl.when(sid == 0)
        def _():
            pltpu.sync_copy(acc_s, sums_out.at[cid])
            pltpu.sync_copy(acc_c, cnt_out.at[cid])

    kern = pl.kernel(
        body,
        out_type=[jax.ShapeDtypeStruct((2, 264, 128), jnp.float32),
                  jax.ShapeDtypeStruct((2, 264, 16), jnp.float32)],
        mesh=_sc_mesh(),
        scratch_types=[
            pltpu.VMEM((chr_, 128), jnp.float32),
            pltpu.VMEM((chr_, 16), jnp.float32),
            pltpu.VMEM((chr_,), jnp.int32),
            pltpu.VMEM_SHARED((264, 128), jnp.float32),
            pltpu.VMEM_SHARED((264, 16), jnp.float32),
            pltpu.SemaphoreType.DMA,
        ],
        compiler_params=pltpu.CompilerParams(use_tc_tiling_on_sc=False),
    )
    return kern(h2new, l2b2)


def _gelu(x):
    return 0.5 * x * (1.0 + jax.lax.erf(x * 0.7071067811865476))


def _rows_spec(blk, w):
    return pl.BlockSpec((blk, w), lambda i: (i, 0))


def _stats_spec():
    return pl.BlockSpec((8, 128), lambda i: (0, 0))


def _stats_merge(st_ref, r, x, blk):
    i = pl.program_id(0)
    mb = jnp.sum(x, axis=0, keepdims=True) * (1.0 / blk)
    m2b = jnp.sum((x - mb) ** 2, axis=0, keepdims=True)

    @pl.when(i == 0)
    def _():
        st_ref[r:r + 1] = mb
        st_ref[r + 1:r + 2] = m2b

    @pl.when(i != 0)
    def _():
        nf = i.astype(jnp.float32) * float(blk)
        mean = st_ref[r:r + 1]
        delta = mb - mean
        tot = nf + float(blk)
        st_ref[r:r + 1] = mean + delta * (float(blk) / tot)
        st_ref[r + 1:r + 2] = (st_ref[r + 1:r + 2] + m2b
                               + delta * delta * (nf * float(blk) / tot))


def _scale_rows_matmul(x_ref, w_ref, s_ref, o_ref):
    h = jnp.dot(x_ref[...], w_ref[...], preferred_element_type=jnp.float32)
    o_ref[...] = s_ref[...] * h


def tc_scale_rows_matmul(x, w, s, blk):
    n, d = x.shape
    return pl.pallas_call(
        _scale_rows_matmul,
        grid=(n // blk,),
        in_specs=[_rows_spec(blk, d), pl.BlockSpec((d, w.shape[1]), lambda i: (0, 0)),
                  _rows_spec(blk, 1)],
        out_specs=_rows_spec(blk, w.shape[1]),
        out_shape=jax.ShapeDtypeStruct((n, w.shape[1]), jnp.float32),
    )(x, w, s)


def _gcn_post(agg_ref, hn_ref, s_ref, b_ref, o_ref):
    o_ref[...] = s_ref[...] * (agg_ref[...] + hn_ref[...]) + b_ref[...]


def tc_gcn_post(agg, hn, s, b, blk):
    n, d = agg.shape
    return pl.pallas_call(
        _gcn_post,
        grid=(n // blk,),
        in_specs=[_rows_spec(blk, d), _rows_spec(blk, d), _rows_spec(blk, 1),
                  pl.BlockSpec((1, d), lambda i: (0, 0))],
        out_specs=_rows_spec(blk, d),
        out_shape=jax.ShapeDtypeStruct((n, d), jnp.float32),
    )(agg, hn, s, b.reshape(1, d))


def _p1_body(blk, hu_ref, hv_ref, w_ref, b_ref, u_ref, es_ref, st_ref):
    hu = hu_ref[...]
    hv = hv_ref[...]
    w = w_ref[...]
    b = b_ref[...]
    c1 = jnp.concatenate([hu, hv], axis=1)
    c2 = jnp.concatenate([hv, hu], axis=1)
    u1 = jnp.dot(c1, w, preferred_element_type=jnp.float32) + b
    u2 = jnp.dot(c2, w, preferred_element_type=jnp.float32) + b
    u = jnp.concatenate([u1, u2], axis=1)
    u_ref[...] = u
    es_ref[...] = hu + hv
    _stats_merge(st_ref, 0, u, blk)


def tc_p1(hu, hv, w_ne0, b_ne0, blk):
    e = hu.shape[0]
    return pl.pallas_call(
        functools.partial(_p1_body, float(blk)),
        grid=(e // blk,),
        in_specs=[_rows_spec(blk, 128), _rows_spec(blk, 128),
                  pl.BlockSpec((256, 64), lambda i: (0, 0)),
                  pl.BlockSpec((1, 64), lambda i: (0, 0))],
        out_specs=[_rows_spec(blk, 128), _rows_spec(blk, 128), _stats_spec()],
        out_shape=[jax.ShapeDtypeStruct((e, 128), jnp.float32),
                   jax.ShapeDtypeStruct((e, 128), jnp.float32),
                   jax.ShapeDtypeStruct((8, 128), jnp.float32)],
    )(hu, hv, w_ne0, b_ne0.reshape(1, 64))


def _p2_body(u_ref, s_ref, t_ref, w_ref, b_ref, o_ref):
    g = _gelu(u_ref[...] * s_ref[...] + t_ref[...])
    w = w_ref[...]
    b = b_ref[...]
    o1 = jnp.dot(g[:, :64], w, preferred_element_type=jnp.float32) + b
    o2 = jnp.dot(g[:, 64:], w, preferred_element_type=jnp.float32) + b
    o_ref[...] = 0.5 * (o1 + o2)


def tc_p2(u, s, t, w_ne1, b_ne1, blk):
    e = u.shape[0]
    return pl.pallas_call(
        _p2_body,
        grid=(e // blk,),
        in_specs=[_rows_spec(blk, 128),
                  pl.BlockSpec((1, 128), lambda i: (0, 0)),
                  pl.BlockSpec((1, 128), lambda i: (0, 0)),
                  pl.BlockSpec((64, 64), lambda i: (0, 0)),
                  pl.BlockSpec((1, 64), lambda i: (0, 0))],
        out_specs=_rows_spec(blk, 64),
        out_shape=jax.ShapeDtypeStruct((e, 64), jnp.float32),
    )(u, s.reshape(1, 128), t.reshape(1, 128), w_ne1, b_ne1.reshape(1, 64))


def _p3_body(blk, he_ref, hf_ref, ee_ref, ef_ref, w_ref, b_ref,
             v_ref, hm_ref, st_ref):
    he = he_ref[...]
    hf = hf_ref[...]
    w = w_ref[...]
    b = b_ref[...]
    v1 = jnp.dot(jnp.concatenate([hf, he], axis=1), w,
                 preferred_element_type=jnp.float32) + b
    v2 = jnp.dot(jnp.concatenate([he, hf], axis=1), w,
                 preferred_element_type=jnp.float32) + b
    v_ref[:, :128] = v1
    v_ref[:, 128:] = v2
    hm_ref[...] = 0.25 * (ee_ref[...] + ef_ref[...])
    _stats_merge(st_ref, 0, v1, blk)
    _stats_merge(st_ref, 2, v2, blk)


def tc_p3(he, hf, ee, ef, w_ef0, b_ef0, blk):
    m = he.shape[0]
    return pl.pallas_call(
        functools.partial(_p3_body, float(blk)),
        grid=(m // blk,),
        in_specs=[_rows_spec(blk, 64), _rows_spec(blk, 64),
                  _rows_spec(blk, 128), _rows_spec(blk, 128),
                  pl.BlockSpec((128, 128), lambda i: (0, 0)),
                  pl.BlockSpec((1, 128), lambda i: (0, 0))],
        out_specs=[_rows_spec(blk, 256), _rows_spec(blk, 128), _stats_spec()],
        out_shape=[jax.ShapeDtypeStruct((m, 256), jnp.float32),
                   jax.ShapeDtypeStruct((m, 128), jnp.float32),
                   jax.ShapeDtypeStruct((8, 128), jnp.float32)],
    )(he, hf, ee, ef, w_ef0, b_ef0.reshape(1, 128))


def _p4_body(blk, v_ref, s_ref, t_ref, w_ref, b_ref, h1_ref, st_ref):
    s = s_ref[...]
    t = t_ref[...]
    g1 = _gelu(v_ref[:, :128] * s[:, :128] + t[:, :128])
    g2 = _gelu(v_ref[:, 128:] * s[:, 128:] + t[:, 128:])
    w = w_ref[...]
    b = b_ref[...]
    h1a = jnp.dot(g1, w, preferred_element_type=jnp.float32) + b
    h1b = jnp.dot(g2, w, preferred_element_type=jnp.float32) + b
    h1 = 0.5 * (h1a + h1b)
    h1_ref[...] = h1
    _stats_merge(st_ref, 0, h1, blk)


def tc_p4(v, s, t, w_ef1, b_ef1, blk):
    m = v.shape[0]
    return pl.pallas_call(
        functools.partial(_p4_body, float(blk)),
        grid=(m // blk,),
        in_specs=[_rows_spec(blk, 256),
                  pl.BlockSpec((1, 256), lambda i: (0, 0)),
                  pl.BlockSpec((1, 256), lambda i: (0, 0)),
                  pl.BlockSpec((128, 128), lambda i: (0, 0)),
                  pl.BlockSpec((1, 128), lambda i: (0, 0))],
        out_specs=[_rows_spec(blk, 128), _stats_spec()],
        out_shape=[jax.ShapeDtypeStruct((m, 128), jnp.float32),
                   jax.ShapeDtypeStruct((8, 128), jnp.float32)],
    )(v, s.reshape(1, 256), t.reshape(1, 256), w_ef1, b_ef1.reshape(1, 128))


def _p5_body(h1_ref, hm_ref, s_ref, t_ref, dinv_ref, w_ref, hn_ref, h2n_ref):
    h1n = hm_ref[...] + jax.nn.relu(h1_ref[...] * s_ref[...] + t_ref[...])
    hn_ref[...] = h1n
    h2 = jnp.dot(h1n, w_ref[...], preferred_element_type=jnp.float32)
    h2n_ref[...] = dinv_ref[...] * h2


def tc_p5(h1, hm, s, t, dinv2, w_gcn2, blk):
    m = h1.shape[0]
    return pl.pallas_call(
        _p5_body,
        grid=(m // blk,),
        in_specs=[_rows_spec(blk, 128), _rows_spec(blk, 128),
                  pl.BlockSpec((1, 128), lambda i: (0, 0)),
                  pl.BlockSpec((1, 128), lambda i: (0, 0)),
                  _rows_spec(blk, 1),
                  pl.BlockSpec((128, 128), lambda i: (0, 0))],
        out_specs=[_rows_spec(blk, 128), _rows_spec(blk, 128)],
        out_shape=[jax.ShapeDtypeStruct((m, 128), jnp.float32),
                   jax.ShapeDtypeStruct((m, 128), jnp.float32)],
    )(h1, hm, s.reshape(1, 128), t.reshape(1, 128), dinv2, w_gcn2)


def _p6_body(blk, agg_ref, h2n_ref, dinv_ref, b_ref, h2_ref, st_ref):
    h2 = dinv_ref[...] * (agg_ref[...] + h2n_ref[...]) + b_ref[...]
    h2_ref[...] = h2
    _stats_merge(st_ref, 0, h2, blk)


def tc_p6(agg, h2n, dinv2, b_gcn2, blk):
    m = h2n.shape[0]
    return pl.pallas_call(
        functools.partial(_p6_body, float(blk)),
        grid=(m // blk,),
        in_specs=[_rows_spec(blk, 128),
                  _rows_spec(blk, 128), _rows_spec(blk, 1),
                  pl.BlockSpec((1, 128), lambda i: (0, 0))],
        out_specs=[_rows_spec(blk, 128), _stats_spec()],
        out_shape=[jax.ShapeDtypeStruct((m, 128), jnp.float32),
                   jax.ShapeDtypeStruct((8, 128), jnp.float32)],
    )(agg, h2n, dinv2, b_gcn2.reshape(1, 128))


def _p7_body(h2_ref, hn_ref, s_ref, t_ref, o_ref):
    o_ref[...] = hn_ref[...] + jax.nn.relu(h2_ref[...] * s_ref[...] + t_ref[...])


def tc_p7(h2, h1n, s, t, blk):
    m = h2.shape[0]
    return pl.pallas_call(
        _p7_body,
        grid=(m // blk,),
        in_specs=[_rows_spec(blk, 128), _rows_spec(blk, 128),
                  pl.BlockSpec((1, 128), lambda i: (0, 0)),
                  pl.BlockSpec((1, 128), lambda i: (0, 0))],
        out_specs=_rows_spec(blk, 128),
        out_shape=jax.ShapeDtypeStruct((m, 128), jnp.float32),
    )(h2, h1n, s.reshape(1, 128), t.reshape(1, 128))


def _ln(x, g, b):
    mu = jnp.mean(x, axis=-1, keepdims=True)
    var = jnp.mean((x - mu) ** 2, axis=-1, keepdims=True)
    return (x - mu) / jnp.sqrt(var + EPS) * g + b


def _p8_body(sums_ref, cnt_ref, w0_ref, b0_ref, g0_ref, t0_ref,
             w1_ref, b1_ref, g1_ref, t1_ref, w2_ref, b2_ref, o_ref):
    hp = sums_ref[...] / jnp.maximum(cnt_ref[...], 1.0)
    h = _gelu(_ln(jnp.dot(hp, w0_ref[...], preferred_element_type=jnp.float32)
                  + b0_ref[...], g0_ref[...], t0_ref[...]))
    h = _gelu(_ln(jnp.dot(h, w1_ref[...], preferred_element_type=jnp.float32)
                  + b1_ref[...], g1_ref[...], t1_ref[...]))
    o_ref[...] = jnp.dot(h, w2_ref[...], preferred_element_type=jnp.float32) \
        + b2_ref[...]


def tc_p8(sums, cnt, w_f0, b_f0, g_ln0, t_ln0, w_f1, b_f1, g_ln1, t_ln1,
          w_f2, b_f2):
    full = lambda shape: pl.BlockSpec(shape, lambda: (0,) * len(shape))
    return pl.pallas_call(
        _p8_body,
        in_specs=[full((256, 128)), full((256, 1)),
                  full((128, 128)), full((1, 128)), full((1, 128)), full((1, 128)),
                  full((128, 128)), full((1, 128)), full((1, 128)), full((1, 128)),
                  full((128, 1)), full((1, 1))],
        out_specs=full((256, 1)),
        out_shape=jax.ShapeDtypeStruct((256, 1), jnp.float32),
    )(sums, cnt.reshape(256, 1), w_f0, b_f0.reshape(1, 128),
      g_ln0.reshape(1, 128), t_ln0.reshape(1, 128), w_f1, b_f1.reshape(1, 128),
      g_ln1.reshape(1, 128), t_ln1.reshape(1, 128), w_f2, b_f2.reshape(1, 1))


def _bn_affine(mu, m2, n, g, b):
    var = m2 / n
    s = g / jnp.sqrt(var + EPS)
    return s, b - mu * s


def kernel(x, edge_index, batch, undirected_edge_mask, l2_node_mapping,
           l2_edge_index, num_graphs, W_gcn1, b_gcn1, W_ne0, b_ne0, g_ne,
           bt_ne, W_ne1, b_ne1, W_ef0, b_ef0, g_ef, bt_ef, W_ef1, b_ef1,
           g_bn1, bt_bn1, g_bn2, bt_bn2, W_gcn2, b_gcn2, W_f0, b_f0, g_ln0,
           bt_ln0, W_f1, b_f1, g_ln1, bt_ln1, W_f2, b_f2):
    n = x.shape[0]
    e = edge_index.shape[1]
    m = l2_node_mapping.shape[1]
    src, dst = edge_index[0], edge_index[1]

    pad1 = 163840 - e
    src1p = jnp.concatenate([src, jnp.zeros((pad1,), src.dtype)])
    dst1p = jnp.concatenate([dst, jnp.full((pad1,), n, dst.dtype)])
    deg1 = jnp.zeros((n,), jnp.float32).at[dst].add(1.0) + 1.0
    dinv1 = jax.lax.rsqrt(deg1).reshape(n, 1)
    h1n = tc_scale_rows_matmul(x, W_gcn1, dinv1, 2000)
    agg1 = sc_gcn_agg1(h1n, src1p, dst1p)
    H0 = tc_gcn_post(agg1, h1n, dinv1, b_gcn1, 2000)

    Hu = H0[src]
    Hv = H0[dst]
    U, Esum, st1 = tc_p1(Hu, Hv, W_ne0, b_ne0, 4000)
    g2 = jnp.concatenate([g_ne, g_ne])
    b2 = jnp.concatenate([bt_ne, bt_ne])
    s_ne, t_ne = _bn_affine(st1[0], st1[1], float(e), g2, b2)
    h_edge = tc_p2(U, s_ne, t_ne, W_ne1, b_ne1, 4000)

    e_idx = l2_node_mapping[0]
    f_idx = l2_node_mapping[1]
    he = h_edge[e_idx]
    hf = h_edge[f_idx]
    ee = Esum[e_idx]
    ef = Esum[f_idx]
    V, H0m, st3 = tc_p3(he, hf, ee, ef, W_ef0, b_ef0, 4000)
    s_ef1, t_ef1 = _bn_affine(st3[0], st3[1], float(m), g_ef, bt_ef)
    s_ef2, t_ef2 = _bn_affine(st3[2], st3[3], float(m), g_ef, bt_ef)
    s_ef = jnp.concatenate([s_ef1, s_ef2])
    t_ef = jnp.concatenate([t_ef1, t_ef2])
    H1, st4 = tc_p4(V, s_ef, t_ef, W_ef1, b_ef1, 4000)
    s_b1, t_b1 = _bn_affine(st4[0], st4[1], float(m), g_bn1, bt_bn1)

    src2, dst2 = l2_edge_index[0], l2_edge_index[1]
    e2 = src2.shape[0]
    pad2 = 409600 - e2
    src2p = jnp.concatenate([src2, jnp.zeros((pad2,), src2.dtype)])
    dst2p = jnp.concatenate([dst2, jnp.full((pad2,), m, dst2.dtype)])
    deg2 = jnp.zeros((m,), jnp.float32).at[dst2].add(1.0) + 1.0
    dinv2 = jax.lax.rsqrt(deg2).reshape(m, 1)
    H1_new, h2n = tc_p5(H1, H0m, s_b1, t_b1, dinv2, W_gcn2, 4000)
    agg2 = jnp.zeros((m, 128), jnp.float32).at[dst2].add(h2n[src2])
    H2, st6 = tc_p6(agg2, h2n, dinv2, b_gcn2, 4000)
    s_b2, t_b2 = _bn_affine(st6[0], st6[1], float(m), g_bn2, bt_bn2)
    H2_new = tc_p7(H2, H1_new, s_b2, t_b2, 4000)

    l2_batch = batch[src[e_idx]].astype(jnp.int32)
    sums_p, cnt_p = sc_segsum(H2_new, l2_batch)
    sums = sums_p[0, :256] + sums_p[1, :256]
    cnt = cnt_p[0, :256, 0] + cnt_p[1, :256, 0]
    return tc_p8(sums, cnt, W_f0, b_f0, g_ln0, bt_ln0, W_f1, b_f1,
                 g_ln1, bt_ln1, W_f2, b_f2)
